# baseline (speedup 1.0000x reference)
import jax
import jax.numpy as jnp
from jax.experimental import pallas as pl
from jax.experimental.pallas import tpu as pltpu

N = 4096
TL = 4
IN_DIM = 128
H0 = 32
H1 = 16

BR_P = 512
BR_C, BC_C = 512, 256
BR_L, BK_L = 512, 512

NC = N // BC_C
NA = (N // BR_C) * NC
NB = N // BR_L

_SLOPE = 0.01


def _lrelu(v):
    return jnp.where(v >= 0, v, _SLOPE * v)


def _issue_dmas(st_hbm, buf, sem, slot, r, c):
    for t in range(TL):
        pltpu.make_async_copy(
            st_hbm.at[pl.ds(r * BR_C, BR_C), t, pl.ds(c * BC_C, BC_C)],
            buf.at[slot, t], sem.at[slot, t]).start()


def _wait_dmas(st_hbm, buf, sem, slot, r, c):
    for t in range(TL):
        pltpu.make_async_copy(
            st_hbm.at[pl.ds(r * BR_C, BR_C), t, pl.ds(c * BC_C, BC_C)],
            buf.at[slot, t], sem.at[slot, t]).wait()


def _mega_body(w_ref, w0_ref, wtT_ref, bt_ref, w1_ref, xt_ref, tvec_ref,
               st_hbm, tv_hbm, prob_hbm,
               buf, sem, sem_out, sup_scr, rs_scr, xw0_scr, b1_scr,
               acc1_scr, b2_scr, tv_scr, prob_scr):
    g = pl.program_id(0)
    f32 = jnp.float32

    @pl.when(g < NB)
    def _():
        xx = (xt_ref[:, 0, :] * w_ref[0, 0] + xt_ref[:, 1, :] * w_ref[1, 0]
              + xt_ref[:, 2, :] * w_ref[2, 0]
              + xt_ref[:, 3, :] * w_ref[3, 0]) * (1.0 / TL)
        xw0_scr[pl.ds(g * BR_P, BR_P), :] = jnp.dot(
            xx, w0_ref[...], preferred_element_type=f32)
        tv = jnp.dot(tvec_ref[...], wtT_ref[...],
                     preferred_element_type=f32) + bt_ref[...]
        tv_scr[pl.ds(g * BR_P, BR_P), :] = jnp.maximum(tv, 0.0)

    @pl.when(g < NA)
    def _():
        r = jax.lax.div(g, NC)
        c = jax.lax.rem(g, NC)
        slot = jax.lax.rem(g, 3)

        @pl.when(g == 0)
        def _():
            _issue_dmas(st_hbm, buf, sem, 0, r, c)
            _issue_dmas(st_hbm, buf, sem, 1, 0, 1)

        @pl.when(g + 2 < NA)
        def _():
            _issue_dmas(st_hbm, buf, sem, jax.lax.rem(g + 2, 3),
                        jax.lax.div(g + 2, NC), jax.lax.rem(g + 2, NC))

        _wait_dmas(st_hbm, buf, sem, slot, r, c)
        tile = buf[slot, 0] * w_ref[0, 0]
        for t in range(1, TL):
            tile = tile + buf[slot, t] * w_ref[t, 0]
        sup_scr[pl.ds(r * BR_C, BR_C), pl.ds(c * BC_C, BC_C)] = (
            tile.astype(jnp.bfloat16))
        part = jnp.sum(tile, axis=1, keepdims=True)

        @pl.when(c == 0)
        def _():
            rs_scr[pl.ds(r * BR_C, BR_C), :] = part

        @pl.when(c > 0)
        def _():
            rs_scr[pl.ds(r * BR_C, BR_C), :] += part

    @pl.when(g == NA)
    def _():
        rs = rs_scr[...]
        dj = jnp.where(rs > 0, jax.lax.rsqrt(rs), 0.0)
        b1_scr[...] = (xw0_scr[...] * dj).astype(jnp.bfloat16)

    @pl.when((g >= NA) & (g < NA + NB))
    def _():
        r2 = g - NA

        def kbody(k, acc):
            s = sup_scr[pl.ds(r2 * BR_L, BR_L), pl.ds(k * BK_L, BK_L)]
            b = b1_scr[pl.ds(k * BK_L, BK_L), :]
            return acc + jnp.dot(s, b, preferred_element_type=f32)

        acc = jax.lax.fori_loop(0, N // BK_L, kbody,
                                jnp.zeros((BR_L, H0), f32))
        acc1_scr[pl.ds(r2 * BR_L, BR_L), :] = acc

    @pl.when(g == NA + NB)
    def _():
        rs = rs_scr[...]
        dsq = jnp.where(rs > 0, 1.0 / rs, 0.0)
        b2_scr[...] = jnp.dot(_lrelu(acc1_scr[...]) * dsq, w1_ref[...],
                              preferred_element_type=f32).astype(jnp.bfloat16)

    @pl.when(g >= NA + NB)
    def _():
        r3 = g - (NA + NB)

        def kbody(k, acc):
            s = sup_scr[pl.ds(r3 * BR_L, BR_L), pl.ds(k * BK_L, BK_L)]
            b = b2_scr[pl.ds(k * BK_L, BK_L), :]
            return acc + jnp.dot(s, b, preferred_element_type=f32)

        acc = jax.lax.fori_loop(0, N // BK_L, kbody,
                                jnp.zeros((BR_L, H1), f32))
        rsr = rs_scr[pl.ds(r3 * BR_L, BR_L), :]
        di = jnp.where(rsr > 0, jax.lax.rsqrt(rsr), 0.0)
        h2 = di * _lrelu(acc)
        m = jnp.max(h2, axis=1, keepdims=True)
        e = jnp.exp(h2 - m)
        prob_scr[pl.ds(r3 * BR_L, BR_L), :] = e / jnp.sum(
            e, axis=1, keepdims=True)

    @pl.when(g == NA + 2 * NB - 1)
    def _():
        pltpu.make_async_copy(tv_scr, tv_hbm, sem_out.at[0]).start()
        pltpu.make_async_copy(prob_scr, prob_hbm, sem_out.at[1]).start()
        pltpu.make_async_copy(tv_scr, tv_hbm, sem_out.at[0]).wait()
        pltpu.make_async_copy(prob_scr, prob_hbm, sem_out.at[1]).wait()


def kernel(x, support, time_vector, w_adj_weight, W0, W1, Wt, bt):
    f32 = jnp.float32
    xt = jnp.transpose(x, (0, 2, 1))
    st = jnp.transpose(support, (0, 2, 1))
    w = w_adj_weight.astype(f32)

    _out = pl.pallas_call(
        _mega_body,
        grid=(NA + 2 * NB,),
        in_specs=[
            pl.BlockSpec(memory_space=pltpu.SMEM),
            pl.BlockSpec((IN_DIM, H0), lambda g: (0, 0)),
            pl.BlockSpec((TL, H1), lambda g: (0, 0)),
            pl.BlockSpec((1, H1), lambda g: (0, 0)),
            pl.BlockSpec((H0, H1), lambda g: (0, 0)),
            pl.BlockSpec((BR_P, TL, IN_DIM),
                         lambda g: (jnp.minimum(g, NB - 1), 0, 0)),
            pl.BlockSpec((BR_P, TL),
                         lambda g: (jnp.minimum(g, NB - 1), 0)),
            pl.BlockSpec(memory_space=pltpu.MemorySpace.HBM),
        ],
        out_specs=[
            pl.BlockSpec(memory_space=pltpu.MemorySpace.HBM),
            pl.BlockSpec(memory_space=pltpu.MemorySpace.HBM),
        ],
        out_shape=[
            jax.ShapeDtypeStruct((N, H1), f32),
            jax.ShapeDtypeStruct((N, H1), f32),
        ],
        scratch_shapes=[
            pltpu.VMEM((3, TL, BR_C, BC_C), f32),
            pltpu.SemaphoreType.DMA((3, TL)),
            pltpu.SemaphoreType.DMA((2,)),
            pltpu.VMEM((N, N), jnp.bfloat16),
            pltpu.VMEM((N, 1), f32),
            pltpu.VMEM((N, H0), f32),
            pltpu.VMEM((N, H0), jnp.bfloat16),
            pltpu.VMEM((N, H0), f32),
            pltpu.VMEM((N, H1), jnp.bfloat16),
            pltpu.VMEM((N, H1), f32),
            pltpu.VMEM((N, H1), f32),
        ],
    )(w, W0, Wt.T, bt.reshape(1, H1), W1, xt, time_vector, st)

    tv, prob = _out
    return (prob, tv)


# speedup vs baseline: 1.0130x; 1.0130x over previous
import jax
import jax.numpy as jnp
from jax.experimental import pallas as pl
from jax.experimental.pallas import tpu as pltpu

N = 4096
TL = 4
IN_DIM = 128
H0 = 32
H1 = 16

BR_P = 512
BR_C, BC_C = 512, 512
BR_L, BK_L = 512, 512

NC = N // BC_C
NA = (N // BR_C) * NC
NB = N // BR_L

_SLOPE = 0.01


def _lrelu(v):
    return jnp.where(v >= 0, v, _SLOPE * v)


def _issue_dmas(st_hbm, buf, sem, slot, r, c):
    for t in range(TL):
        pltpu.make_async_copy(
            st_hbm.at[pl.ds(r * BR_C, BR_C), t, pl.ds(c * BC_C, BC_C)],
            buf.at[slot, t], sem.at[slot, t]).start()


def _wait_dmas(st_hbm, buf, sem, slot, r, c):
    for t in range(TL):
        pltpu.make_async_copy(
            st_hbm.at[pl.ds(r * BR_C, BR_C), t, pl.ds(c * BC_C, BC_C)],
            buf.at[slot, t], sem.at[slot, t]).wait()


def _mega_body(w_ref, w0_ref, wtT_ref, bt_ref, w1_ref, xt_ref, tvec_ref,
               st_hbm, tv_hbm, prob_hbm,
               buf, sem, sem_out, sup_scr, rs_scr, xw0_scr, b1_scr,
               acc1_scr, b2_scr, tv_scr, prob_scr):
    g = pl.program_id(0)
    f32 = jnp.float32

    @pl.when(g < NB)
    def _():
        xx = (xt_ref[:, 0, :] * w_ref[0, 0] + xt_ref[:, 1, :] * w_ref[1, 0]
              + xt_ref[:, 2, :] * w_ref[2, 0]
              + xt_ref[:, 3, :] * w_ref[3, 0]) * (1.0 / TL)
        xw0_scr[pl.ds(g * BR_P, BR_P), :] = jnp.dot(
            xx, w0_ref[...], preferred_element_type=f32)
        tv = jnp.dot(tvec_ref[...], wtT_ref[...],
                     preferred_element_type=f32) + bt_ref[...]
        tv_scr[pl.ds(g * BR_P, BR_P), :] = jnp.maximum(tv, 0.0)

    @pl.when(g < NA)
    def _():
        r = jax.lax.div(g, NC)
        c = jax.lax.rem(g, NC)
        slot = jax.lax.rem(g, 2)

        @pl.when(g == 0)
        def _():
            _issue_dmas(st_hbm, buf, sem, 0, r, c)

        @pl.when(g + 1 < NA)
        def _():
            _issue_dmas(st_hbm, buf, sem, 1 - slot,
                        jax.lax.div(g + 1, NC), jax.lax.rem(g + 1, NC))

        _wait_dmas(st_hbm, buf, sem, slot, r, c)
        tile = buf[slot, 0] * w_ref[0, 0]
        for t in range(1, TL):
            tile = tile + buf[slot, t] * w_ref[t, 0]
        sup_scr[pl.ds(r * BR_C, BR_C), pl.ds(c * BC_C, BC_C)] = (
            tile.astype(jnp.bfloat16))
        part = jnp.sum(tile, axis=1, keepdims=True)

        @pl.when(c == 0)
        def _():
            rs_scr[pl.ds(r * BR_C, BR_C), :] = part

        @pl.when(c > 0)
        def _():
            rs_scr[pl.ds(r * BR_C, BR_C), :] += part

    @pl.when(g == NA)
    def _():
        rs = rs_scr[...]
        dj = jnp.where(rs > 0, jax.lax.rsqrt(rs), 0.0)
        b1_scr[...] = (xw0_scr[...] * dj).astype(jnp.bfloat16)

    @pl.when((g >= NA) & (g < NA + NB))
    def _():
        r2 = g - NA

        def kbody(k, acc):
            s = sup_scr[pl.ds(r2 * BR_L, BR_L), pl.ds(k * BK_L, BK_L)]
            b = b1_scr[pl.ds(k * BK_L, BK_L), :]
            return acc + jnp.dot(s, b, preferred_element_type=f32)

        acc = jax.lax.fori_loop(0, N // BK_L, kbody,
                                jnp.zeros((BR_L, H0), f32))
        acc1_scr[pl.ds(r2 * BR_L, BR_L), :] = acc

    @pl.when(g == NA + NB)
    def _():
        rs = rs_scr[...]
        dsq = jnp.where(rs > 0, 1.0 / rs, 0.0)
        b2_scr[...] = jnp.dot(_lrelu(acc1_scr[...]) * dsq, w1_ref[...],
                              preferred_element_type=f32).astype(jnp.bfloat16)

    @pl.when(g >= NA + NB)
    def _():
        r3 = g - (NA + NB)

        def kbody(k, acc):
            s = sup_scr[pl.ds(r3 * BR_L, BR_L), pl.ds(k * BK_L, BK_L)]
            b = b2_scr[pl.ds(k * BK_L, BK_L), :]
            return acc + jnp.dot(s, b, preferred_element_type=f32)

        acc = jax.lax.fori_loop(0, N // BK_L, kbody,
                                jnp.zeros((BR_L, H1), f32))
        rsr = rs_scr[pl.ds(r3 * BR_L, BR_L), :]
        di = jnp.where(rsr > 0, jax.lax.rsqrt(rsr), 0.0)
        h2 = di * _lrelu(acc)
        m = jnp.max(h2, axis=1, keepdims=True)
        e = jnp.exp(h2 - m)
        prob_scr[pl.ds(r3 * BR_L, BR_L), :] = e / jnp.sum(
            e, axis=1, keepdims=True)

    @pl.when(g == NA + 2 * NB - 1)
    def _():
        pltpu.make_async_copy(tv_scr, tv_hbm, sem_out.at[0]).start()
        pltpu.make_async_copy(prob_scr, prob_hbm, sem_out.at[1]).start()
        pltpu.make_async_copy(tv_scr, tv_hbm, sem_out.at[0]).wait()
        pltpu.make_async_copy(prob_scr, prob_hbm, sem_out.at[1]).wait()


def kernel(x, support, time_vector, w_adj_weight, W0, W1, Wt, bt):
    f32 = jnp.float32
    xt = jnp.transpose(x, (0, 2, 1))
    st = jnp.transpose(support, (0, 2, 1))
    w = w_adj_weight.astype(f32)

    _out = pl.pallas_call(
        _mega_body,
        grid=(NA + 2 * NB,),
        in_specs=[
            pl.BlockSpec(memory_space=pltpu.SMEM),
            pl.BlockSpec((IN_DIM, H0), lambda g: (0, 0)),
            pl.BlockSpec((TL, H1), lambda g: (0, 0)),
            pl.BlockSpec((1, H1), lambda g: (0, 0)),
            pl.BlockSpec((H0, H1), lambda g: (0, 0)),
            pl.BlockSpec((BR_P, TL, IN_DIM),
                         lambda g: (jnp.minimum(g, NB - 1), 0, 0)),
            pl.BlockSpec((BR_P, TL),
                         lambda g: (jnp.minimum(g, NB - 1), 0)),
            pl.BlockSpec(memory_space=pltpu.MemorySpace.HBM),
        ],
        out_specs=[
            pl.BlockSpec(memory_space=pltpu.MemorySpace.HBM),
            pl.BlockSpec(memory_space=pltpu.MemorySpace.HBM),
        ],
        out_shape=[
            jax.ShapeDtypeStruct((N, H1), f32),
            jax.ShapeDtypeStruct((N, H1), f32),
        ],
        scratch_shapes=[
            pltpu.VMEM((2, TL, BR_C, BC_C), f32),
            pltpu.SemaphoreType.DMA((2, TL)),
            pltpu.SemaphoreType.DMA((2,)),
            pltpu.VMEM((N, N), jnp.bfloat16),
            pltpu.VMEM((N, 1), f32),
            pltpu.VMEM((N, H0), f32),
            pltpu.VMEM((N, H0), jnp.bfloat16),
            pltpu.VMEM((N, H0), f32),
            pltpu.VMEM((N, H1), jnp.bfloat16),
            pltpu.VMEM((N, H1), f32),
            pltpu.VMEM((N, H1), f32),
        ],
    )(w, W0, Wt.T, bt.reshape(1, H1), W1, xt, time_vector, st)

    tv, prob = _out
    return (prob, tv)


# speedup vs baseline: 1.0691x; 1.0554x over previous
import jax
import jax.numpy as jnp
from jax.experimental import pallas as pl
from jax.experimental.pallas import tpu as pltpu

N = 4096
TL = 4
IN_DIM = 128
H0 = 32
H1 = 16

BR_P = 512
BR_C, BC_C = 512, 512
BR_L, BK_L = 512, 512

NC = N // BC_C
NA = (N // BR_C) * NC
NB = N // BR_L

_SLOPE = 0.01


def _lrelu(v):
    return jnp.where(v >= 0, v, _SLOPE * v)


def _issue_dmas(st_hbm, buf, sem, slot, r, c):
    for t in range(TL):
        pltpu.make_async_copy(
            st_hbm.at[pl.ds(r * BR_C, BR_C), t, pl.ds(c * BC_C, BC_C)],
            buf.at[slot, t], sem.at[slot, t]).start()


def _wait_dmas(st_hbm, buf, sem, slot, r, c):
    for t in range(TL):
        pltpu.make_async_copy(
            st_hbm.at[pl.ds(r * BR_C, BR_C), t, pl.ds(c * BC_C, BC_C)],
            buf.at[slot, t], sem.at[slot, t]).wait()


def _prep_body(xt_ref, w_ref, w0_ref, tvec_ref, wtT_ref, bt_ref,
               xw0_ref, tv_ref):
    f32 = jnp.float32
    xx = (xt_ref[:, 0, :] * w_ref[0, 0] + xt_ref[:, 1, :] * w_ref[1, 0]
          + xt_ref[:, 2, :] * w_ref[2, 0]
          + xt_ref[:, 3, :] * w_ref[3, 0]) * (1.0 / TL)
    xw0_ref[...] = jnp.dot(xx, w0_ref[...], preferred_element_type=f32)
    tv = jnp.dot(tvec_ref[...], wtT_ref[...],
                 preferred_element_type=f32) + bt_ref[...]
    tv_ref[...] = jnp.maximum(tv, 0.0)


def _mega_body(w_ref, w1_ref, xw0_in, st_hbm, prob_hbm,
               buf, sem, sem_out, sup_scr, rs_scr, b1_scr,
               acc1_scr, b2_scr, prob_scr):
    g = pl.program_id(0)
    f32 = jnp.float32

    @pl.when(g < NA)
    def _():
        r = jax.lax.div(g, NC)
        c = jax.lax.rem(g, NC)
        slot = jax.lax.rem(g, 3)

        @pl.when(g == 0)
        def _():
            _issue_dmas(st_hbm, buf, sem, 0, r, c)
            _issue_dmas(st_hbm, buf, sem, 1, 0, 1)

        @pl.when(g + 2 < NA)
        def _():
            _issue_dmas(st_hbm, buf, sem, jax.lax.rem(g + 2, 3),
                        jax.lax.div(g + 2, NC), jax.lax.rem(g + 2, NC))

        _wait_dmas(st_hbm, buf, sem, slot, r, c)
        tile = buf[slot, 0] * w_ref[0, 0]
        for t in range(1, TL):
            tile = tile + buf[slot, t] * w_ref[t, 0]
        sup_scr[pl.ds(r * BR_C, BR_C), pl.ds(c * BC_C, BC_C)] = (
            tile.astype(jnp.bfloat16))
        part = jnp.sum(tile, axis=1, keepdims=True)

        @pl.when(c == 0)
        def _():
            rs_scr[pl.ds(r * BR_C, BR_C), :] = part

        @pl.when(c > 0)
        def _():
            rs_scr[pl.ds(r * BR_C, BR_C), :] += part

    @pl.when(g == NA)
    def _():
        rs = rs_scr[...]
        dj = jnp.where(rs > 0, jax.lax.rsqrt(rs), 0.0)
        b1_scr[...] = (xw0_in[...] * dj).astype(jnp.bfloat16)

    @pl.when((g >= NA) & (g < NA + NB))
    def _():
        r2 = g - NA

        def kbody(k, acc):
            s = sup_scr[pl.ds(r2 * BR_L, BR_L), pl.ds(k * BK_L, BK_L)]
            b = b1_scr[pl.ds(k * BK_L, BK_L), :]
            return acc + jnp.dot(s, b, preferred_element_type=f32)

        acc = jax.lax.fori_loop(0, N // BK_L, kbody,
                                jnp.zeros((BR_L, H0), f32))
        acc1_scr[pl.ds(r2 * BR_L, BR_L), :] = acc

    @pl.when(g == NA + NB)
    def _():
        rs = rs_scr[...]
        dsq = jnp.where(rs > 0, 1.0 / rs, 0.0)
        b2_scr[...] = jnp.dot(_lrelu(acc1_scr[...]) * dsq, w1_ref[...],
                              preferred_element_type=f32).astype(jnp.bfloat16)

    @pl.when(g >= NA + NB)
    def _():
        r3 = g - (NA + NB)

        def kbody(k, acc):
            s = sup_scr[pl.ds(r3 * BR_L, BR_L), pl.ds(k * BK_L, BK_L)]
            b = b2_scr[pl.ds(k * BK_L, BK_L), :]
            return acc + jnp.dot(s, b, preferred_element_type=f32)

        acc = jax.lax.fori_loop(0, N // BK_L, kbody,
                                jnp.zeros((BR_L, H1), f32))
        rsr = rs_scr[pl.ds(r3 * BR_L, BR_L), :]
        di = jnp.where(rsr > 0, jax.lax.rsqrt(rsr), 0.0)
        h2 = di * _lrelu(acc)
        m = jnp.max(h2, axis=1, keepdims=True)
        e = jnp.exp(h2 - m)
        prob_scr[pl.ds(r3 * BR_L, BR_L), :] = e / jnp.sum(
            e, axis=1, keepdims=True)

    @pl.when(g == NA + 2 * NB - 1)
    def _():
        pltpu.make_async_copy(prob_scr, prob_hbm, sem_out.at[0]).start()
        pltpu.make_async_copy(prob_scr, prob_hbm, sem_out.at[0]).wait()


def kernel(x, support, time_vector, w_adj_weight, W0, W1, Wt, bt):
    f32 = jnp.float32
    xt = jnp.transpose(x, (0, 2, 1))
    st = jnp.transpose(support, (0, 2, 1))
    w = w_adj_weight.astype(f32)

    xw0, tv = pl.pallas_call(
        _prep_body,
        grid=(N // BR_P,),
        in_specs=[
            pl.BlockSpec((BR_P, TL, IN_DIM), lambda r: (r, 0, 0)),
            pl.BlockSpec(memory_space=pltpu.SMEM),
            pl.BlockSpec((IN_DIM, H0), lambda r: (0, 0)),
            pl.BlockSpec((BR_P, TL), lambda r: (r, 0)),
            pl.BlockSpec((TL, H1), lambda r: (0, 0)),
            pl.BlockSpec((1, H1), lambda r: (0, 0)),
        ],
        out_specs=[
            pl.BlockSpec((BR_P, H0), lambda r: (r, 0)),
            pl.BlockSpec((BR_P, H1), lambda r: (r, 0)),
        ],
        out_shape=[
            jax.ShapeDtypeStruct((N, H0), f32),
            jax.ShapeDtypeStruct((N, H1), f32),
        ],
    )(xt, w, W0, time_vector, Wt.T, bt.reshape(1, H1))

    prob = pl.pallas_call(
        _mega_body,
        grid=(NA + 2 * NB,),
        in_specs=[
            pl.BlockSpec(memory_space=pltpu.SMEM),
            pl.BlockSpec((H0, H1), lambda g: (0, 0)),
            pl.BlockSpec((N, H0), lambda g: (0, 0)),
            pl.BlockSpec(memory_space=pltpu.MemorySpace.HBM),
        ],
        out_specs=pl.BlockSpec(memory_space=pltpu.MemorySpace.HBM),
        out_shape=jax.ShapeDtypeStruct((N, H1), f32),
        scratch_shapes=[
            pltpu.VMEM((3, TL, BR_C, BC_C), f32),
            pltpu.SemaphoreType.DMA((3, TL)),
            pltpu.SemaphoreType.DMA((1,)),
            pltpu.VMEM((N, N), jnp.bfloat16),
            pltpu.VMEM((N, 1), f32),
            pltpu.VMEM((N, H0), jnp.bfloat16),
            pltpu.VMEM((N, H0), f32),
            pltpu.VMEM((N, H1), jnp.bfloat16),
            pltpu.VMEM((N, H1), f32),
        ],
    )(w, W1, xw0, st)

    return (prob, tv)
